# Initial kernel scaffold; baseline (speedup 1.0000x reference)
import jax
import jax.numpy as jnp
from jax.experimental import pallas as pl


def kernel(inputs, emb, conv_w, conv_b, fc_w, fc_b):
    raise NotImplementedError("write your pallas kernel here")


# speedup vs baseline: 9.5407x; 9.5407x over previous
import functools

import jax
import jax.numpy as jnp
from jax import lax
from jax.experimental import pallas as pl
from jax.experimental.pallas import tpu as pltpu

B, L, V, D, F, K, KW = 1024, 200, 100000, 10, 10, 50, 3
DP = 16
S = 2 * B
GS = 8
G = S // GS
N = S * L
LV = L - KW + 1
NEG = float(jnp.finfo(jnp.float32).min)


def _sc_gather(table, idx):
    from jax.experimental.pallas import tpu_sc as plsc

    info = plsc.get_sparse_core_info()
    NC, NS = info.num_cores, info.num_subcores
    NW = NC * NS
    b_per_w = N // NW
    CH = 3200
    mesh = plsc.VectorSubcoreMesh(core_axis_name="c", subcore_axis_name="s")

    @functools.partial(
        pl.kernel,
        mesh=mesh,
        out_type=jax.ShapeDtypeStruct((N, DP), jnp.float32),
        scratch_types=[
            pltpu.VMEM((CH,), jnp.int32),
            pltpu.VMEM((CH, DP), jnp.float32),
            pltpu.SemaphoreType.DMA,
        ],
        compiler_params=pltpu.CompilerParams(use_tc_tiling_on_sc=False),
    )
    def k(table_hbm, idx_hbm, out_hbm, idx_v, rows_v, sem):
        wid = lax.axis_index("s") * NC + lax.axis_index("c")
        base = wid * b_per_w
        for c in range(b_per_w // CH):
            off = base + c * CH
            pltpu.sync_copy(idx_hbm.at[pl.ds(off, CH)], idx_v)
            pltpu.async_copy(table_hbm.at[idx_v], rows_v, sem).wait()
            pltpu.sync_copy(rows_v, out_hbm.at[pl.ds(off, CH)])

    return k(table, idx)


def _tc_body(x_ref, w_ref, bias_ref, wfc_ref, fcb_ref, o_ref, vals_ref):
    X = x_ref[0]
    z1 = jnp.zeros((1, 128), jnp.float32)
    X1 = jnp.concatenate([X[1:], z1], axis=0)
    X2 = jnp.concatenate([X[2:], z1, z1], axis=0)
    Y = (jnp.dot(X, w_ref[0], preferred_element_type=jnp.float32)
         + jnp.dot(X1, w_ref[1], preferred_element_type=jnp.float32)
         + jnp.dot(X2, w_ref[2], preferred_element_type=jnp.float32))
    Y = Y + bias_ref[0:1]
    tio = lax.broadcasted_iota(jnp.int32, (L, 128), 0)
    Y = jnp.where(tio < LV, Y, NEG)

    def step(k, Y):
        m = jnp.max(Y, axis=0, keepdims=True)
        hit = Y == m
        first = jnp.min(jnp.where(hit, tio, L), axis=0, keepdims=True)
        vals_ref[pl.ds(k, 1)] = m
        return jnp.where(tio == first, NEG, Y)

    lax.fori_loop(0, K, step, Y, unroll=True)
    V50 = vals_ref[0:K]
    out = jnp.dot(V50, wfc_ref[...], preferred_element_type=jnp.float32)
    o_ref[0] = jnp.maximum(out + fcb_ref[0:1], 0.0)


def _tc_stage(xg, wbig, bias_t, wfc, fcb):
    return pl.pallas_call(
        _tc_body,
        grid=(G,),
        in_specs=[
            pl.BlockSpec((1, L, 128), lambda g: (g, 0, 0)),
            pl.BlockSpec((KW, 128, 128), lambda g: (0, 0, 0)),
            pl.BlockSpec((8, 128), lambda g: (0, 0)),
            pl.BlockSpec((128, 16), lambda g: (0, 0)),
            pl.BlockSpec((8, 16), lambda g: (0, 0)),
        ],
        out_specs=pl.BlockSpec((1, K, 16), lambda g: (g, 0, 0)),
        out_shape=jax.ShapeDtypeStruct((G, K, 16), jnp.float32),
        scratch_shapes=[pltpu.VMEM((K + 6, 128), jnp.float32)],
    )(xg, wbig, bias_t, wfc, fcb)


def kernel(inputs, emb, conv_w, conv_b, fc_w, fc_b):
    seqs = inputs.reshape(S, L).astype(jnp.int32)
    idx = seqs.reshape(G, GS, L).transpose(0, 2, 1).reshape(-1)
    table = jnp.concatenate(
        [emb, jnp.zeros((V, DP - D), jnp.float32)], axis=1)

    gathered = _sc_gather(table, idx)
    xg = gathered.reshape(G, L, GS * DP)

    eye8 = jnp.eye(GS, dtype=jnp.float32)
    wp = jnp.zeros((KW, DP, DP), jnp.float32).at[:, :D, :D].set(conv_w)
    wbig = jnp.stack([jnp.kron(eye8, wp[w]) for w in range(KW)])
    bias_t = jnp.tile(
        jnp.concatenate([conv_b, jnp.zeros((DP - F,), jnp.float32)]), (GS,))
    bias_t = jnp.broadcast_to(bias_t, (8, 128))
    wfc = jnp.kron(eye8, jnp.zeros((DP, 2), jnp.float32).at[:D].set(fc_w))
    fcb = jnp.broadcast_to(jnp.tile(fc_b, (GS,)), (8, 16))

    out = _tc_stage(xg, wbig, bias_t, wfc, fcb)

    out = out.reshape(G, K, GS, 2).transpose(0, 2, 1, 3)
    out = out.reshape(B, 2, K, 2).reshape(B, 2 * K, 2)
    return out


# speedup vs baseline: 13.6346x; 1.4291x over previous
import functools

import jax
import jax.numpy as jnp
from jax import lax
from jax.experimental import pallas as pl
from jax.experimental.pallas import tpu as pltpu

B, L, V, D, F, K, KW = 1024, 200, 100000, 10, 10, 50, 3
DP = 16
S = 2 * B
GS = 8
G = S // GS
N = S * L
LV = L - KW + 1
NEG = float(jnp.finfo(jnp.float32).min)


def _sc_gather(table, idx):
    from jax.experimental.pallas import tpu_sc as plsc

    info = plsc.get_sparse_core_info()
    NC, NS = info.num_cores, info.num_subcores
    NW = NC * NS
    b_per_w = N // NW
    CH = 3200
    mesh = plsc.VectorSubcoreMesh(core_axis_name="c", subcore_axis_name="s")

    @functools.partial(
        pl.kernel,
        mesh=mesh,
        out_type=jax.ShapeDtypeStruct((N, DP), jnp.float32),
        scratch_types=[
            pltpu.VMEM((CH,), jnp.int32),
            pltpu.VMEM((CH, DP), jnp.float32),
            pltpu.SemaphoreType.DMA,
        ],
        compiler_params=pltpu.CompilerParams(use_tc_tiling_on_sc=False),
    )
    def k(table_hbm, idx_hbm, out_hbm, idx_v, rows_v, sem):
        wid = lax.axis_index("s") * NC + lax.axis_index("c")
        base = wid * b_per_w
        for c in range(b_per_w // CH):
            off = base + c * CH
            pltpu.sync_copy(idx_hbm.at[pl.ds(off, CH)], idx_v)
            pltpu.async_copy(table_hbm.at[idx_v], rows_v, sem).wait()
            pltpu.sync_copy(rows_v, out_hbm.at[pl.ds(off, CH)])

    return k(table, idx)


def _tc_body(x_ref, w_ref, bias_ref, wfc_ref, fcb_ref, o_ref, vals_ref):
    INT_MIN = jnp.int32(-2**31)
    X = x_ref[0]
    z1 = jnp.zeros((1, 128), jnp.float32)
    X1 = jnp.concatenate([X[1:], z1], axis=0)
    X2 = jnp.concatenate([X[2:], z1, z1], axis=0)
    Y = (jnp.dot(X, w_ref[0], preferred_element_type=jnp.float32)
         + jnp.dot(X1, w_ref[1], preferred_element_type=jnp.float32)
         + jnp.dot(X2, w_ref[2], preferred_element_type=jnp.float32))
    Y = Y + bias_ref[0:1]
    s = lax.bitcast_convert_type(Y, jnp.int32)
    key = jnp.where(s >= 0, s, (~s) ^ INT_MIN)
    tio = lax.broadcasted_iota(jnp.int32, (L, 128), 0)
    key = (key & ~jnp.int32(255)) | (jnp.int32(255) - tio)
    key = jnp.where(tio < LV, key, INT_MIN)

    def step(k, key):
        m = jnp.max(key, axis=0, keepdims=True)
        vals_ref[pl.ds(k, 1)] = m
        return jnp.where(key == m, INT_MIN, key)

    lax.fori_loop(0, K, step, key, unroll=True)
    k50 = vals_ref[0:K] & ~jnp.int32(255)
    s50 = jnp.where(k50 >= 0, k50, ~(k50 ^ INT_MIN))
    V50 = lax.bitcast_convert_type(s50, jnp.float32)
    out = jnp.dot(V50, wfc_ref[...], preferred_element_type=jnp.float32)
    o_ref[0] = jnp.maximum(out + fcb_ref[0:1], 0.0)


def _tc_stage(xg, wbig, bias_t, wfc, fcb):
    return pl.pallas_call(
        _tc_body,
        grid=(G,),
        in_specs=[
            pl.BlockSpec((1, L, 128), lambda g: (g, 0, 0)),
            pl.BlockSpec((KW, 128, 128), lambda g: (0, 0, 0)),
            pl.BlockSpec((8, 128), lambda g: (0, 0)),
            pl.BlockSpec((128, 16), lambda g: (0, 0)),
            pl.BlockSpec((8, 16), lambda g: (0, 0)),
        ],
        out_specs=pl.BlockSpec((1, K, 16), lambda g: (g, 0, 0)),
        out_shape=jax.ShapeDtypeStruct((G, K, 16), jnp.float32),
        scratch_shapes=[pltpu.VMEM((K + 6, 128), jnp.int32)],
    )(xg, wbig, bias_t, wfc, fcb)


def kernel(inputs, emb, conv_w, conv_b, fc_w, fc_b):
    seqs = inputs.reshape(S, L).astype(jnp.int32)
    idx = seqs.reshape(G, GS, L).transpose(0, 2, 1).reshape(-1)
    table = jnp.concatenate(
        [emb, jnp.zeros((V, DP - D), jnp.float32)], axis=1)

    gathered = _sc_gather(table, idx)
    xg = gathered.reshape(G, L, GS * DP)

    eye8 = jnp.eye(GS, dtype=jnp.float32)
    wp = jnp.zeros((KW, DP, DP), jnp.float32).at[:, :D, :D].set(conv_w)
    wbig = jnp.stack([jnp.kron(eye8, wp[w]) for w in range(KW)])
    bias_t = jnp.tile(
        jnp.concatenate([conv_b, jnp.zeros((DP - F,), jnp.float32)]), (GS,))
    bias_t = jnp.broadcast_to(bias_t, (8, 128))
    wfc = jnp.kron(eye8, jnp.zeros((DP, 2), jnp.float32).at[:D].set(fc_w))
    fcb = jnp.broadcast_to(jnp.tile(fc_b, (GS,)), (8, 16))

    out = _tc_stage(xg, wbig, bias_t, wfc, fcb)

    out = out.reshape(G, K, GS, 2).transpose(0, 2, 1, 3)
    out = out.reshape(B, 2, K, 2).reshape(B, 2 * K, 2)
    return out


# speedup vs baseline: 15.7067x; 1.1520x over previous
import functools

import jax
import jax.numpy as jnp
from jax import lax
from jax.experimental import pallas as pl
from jax.experimental.pallas import tpu as pltpu

B, L, V, D, F, K, KW = 1024, 200, 100000, 10, 10, 50, 3
DP = 16
S = 2 * B
GS = 8
G = S // GS
N = S * L
LV = L - KW + 1
NEG = float(jnp.finfo(jnp.float32).min)


def _sc_gather(table, idx):
    from jax.experimental.pallas import tpu_sc as plsc

    info = plsc.get_sparse_core_info()
    NC, NS = info.num_cores, info.num_subcores
    NW = NC * NS
    b_per_w = N // NW
    CH = 3200
    mesh = plsc.VectorSubcoreMesh(core_axis_name="c", subcore_axis_name="s")

    @functools.partial(
        pl.kernel,
        mesh=mesh,
        out_type=jax.ShapeDtypeStruct((N, DP), jnp.float32),
        scratch_types=[
            pltpu.VMEM((CH,), jnp.int32),
            pltpu.VMEM((CH, DP), jnp.float32),
            pltpu.SemaphoreType.DMA,
        ],
        compiler_params=pltpu.CompilerParams(use_tc_tiling_on_sc=False),
    )
    def k(table_hbm, idx_hbm, out_hbm, idx_v, rows_v, sem):
        wid = lax.axis_index("s") * NC + lax.axis_index("c")
        base = wid * b_per_w
        for c in range(b_per_w // CH):
            off = base + c * CH
            pltpu.sync_copy(idx_hbm.at[pl.ds(off, CH)], idx_v)
            pltpu.async_copy(table_hbm.at[idx_v], rows_v, sem).wait()
            pltpu.sync_copy(rows_v, out_hbm.at[pl.ds(off, CH)])

    return k(table, idx)


GPB = 4


def _tc_body(x_ref, w_ref, bias_ref, wfc_ref, fcb_ref, o_ref, vals_ref):
    INT_MIN = jnp.int32(-2**31)
    X = x_ref[...]
    zg = jnp.zeros((GPB, 1, 128), jnp.float32)
    X1 = jnp.concatenate([X[:, 1:], zg], axis=1)
    X2 = jnp.concatenate([X[:, 2:], zg, zg], axis=1)

    def mm(a, w):
        r = jnp.dot(a.reshape(GPB * L, 128), w,
                    preferred_element_type=jnp.float32)
        return r.reshape(GPB, L, 128)

    Y = mm(X, w_ref[0]) + mm(X1, w_ref[1]) + mm(X2, w_ref[2])
    Y = Y + bias_ref[0:1]
    s = lax.bitcast_convert_type(Y, jnp.int32)
    key = jnp.where(s >= 0, s, (~s) ^ INT_MIN)
    tio = lax.broadcasted_iota(jnp.int32, (GPB, L, 128), 1)
    key = (key & ~jnp.int32(255)) | (jnp.int32(255) - tio)
    key = jnp.where(tio < LV, key, INT_MIN)

    def step(k, key):
        m = jnp.max(key, axis=1, keepdims=True)
        vals_ref[:, pl.ds(k, 1)] = m
        return jnp.where(key == m, INT_MIN, key)

    lax.fori_loop(0, K, step, key, unroll=True)
    k50 = (vals_ref[:, 0:K] & ~jnp.int32(255)) | jnp.int32(128)
    s50 = jnp.where(k50 >= 0, k50, ~(k50 ^ INT_MIN))
    V50 = lax.bitcast_convert_type(s50, jnp.float32)
    out = jnp.dot(V50.reshape(GPB * K, 128), wfc_ref[...],
                  preferred_element_type=jnp.float32).reshape(GPB, K, 16)
    o_ref[...] = jnp.maximum(out + fcb_ref[0:1], 0.0)


def _tc_stage(xg, wbig, bias_t, wfc, fcb):
    return pl.pallas_call(
        _tc_body,
        grid=(G // GPB,),
        in_specs=[
            pl.BlockSpec((GPB, L, 128), lambda g: (g, 0, 0)),
            pl.BlockSpec((KW, 128, 128), lambda g: (0, 0, 0)),
            pl.BlockSpec((8, 128), lambda g: (0, 0)),
            pl.BlockSpec((128, 16), lambda g: (0, 0)),
            pl.BlockSpec((8, 16), lambda g: (0, 0)),
        ],
        out_specs=pl.BlockSpec((GPB, K, 16), lambda g: (g, 0, 0)),
        out_shape=jax.ShapeDtypeStruct((G, K, 16), jnp.float32),
        scratch_shapes=[pltpu.VMEM((GPB, K + 6, 128), jnp.int32)],
    )(xg, wbig, bias_t, wfc, fcb)


def kernel(inputs, emb, conv_w, conv_b, fc_w, fc_b):
    seqs = inputs.reshape(S, L).astype(jnp.int32)
    idx = seqs.reshape(G, GS, L).transpose(0, 2, 1).reshape(-1)
    table = jnp.concatenate(
        [emb, jnp.zeros((V, DP - D), jnp.float32)], axis=1)

    gathered = _sc_gather(table, idx)
    xg = gathered.reshape(G, L, GS * DP)

    eye8 = jnp.eye(GS, dtype=jnp.float32)
    wp = jnp.zeros((KW, DP, DP), jnp.float32).at[:, :D, :D].set(conv_w)
    wbig = jnp.stack([jnp.kron(eye8, wp[w]) for w in range(KW)])
    bias_t = jnp.tile(
        jnp.concatenate([conv_b, jnp.zeros((DP - F,), jnp.float32)]), (GS,))
    bias_t = jnp.broadcast_to(bias_t, (8, 128))
    wfc = jnp.kron(eye8, jnp.zeros((DP, 2), jnp.float32).at[:D].set(fc_w))
    fcb = jnp.broadcast_to(jnp.tile(fc_b, (GS,)), (8, 16))

    out = _tc_stage(xg, wbig, bias_t, wfc, fcb)

    out = out.reshape(G, K, GS, 2).transpose(0, 2, 1, 3)
    out = out.reshape(B, 2, K, 2).reshape(B, 2 * K, 2)
    return out


# speedup vs baseline: 17.2158x; 1.0961x over previous
import functools

import jax
import jax.numpy as jnp
from jax import lax
from jax.experimental import pallas as pl
from jax.experimental.pallas import tpu as pltpu

B, L, V, D, F, K, KW = 1024, 200, 100000, 10, 10, 50, 3
DP = 16
S = 2 * B
GS = 8
G = S // GS
N = S * L
LV = L - KW + 1
NEG = float(jnp.finfo(jnp.float32).min)


def _sc_gather(table, idx):
    from jax.experimental.pallas import tpu_sc as plsc

    info = plsc.get_sparse_core_info()
    NC, NS = info.num_cores, info.num_subcores
    NW = NC * NS
    b_per_w = N // NW
    CH = 3200
    mesh = plsc.VectorSubcoreMesh(core_axis_name="c", subcore_axis_name="s")

    @functools.partial(
        pl.kernel,
        mesh=mesh,
        out_type=jax.ShapeDtypeStruct((N, DP), jnp.float32),
        scratch_types=[
            pltpu.VMEM((CH,), jnp.int32),
            pltpu.VMEM((CH, DP), jnp.float32),
            pltpu.SemaphoreType.DMA,
        ],
        compiler_params=pltpu.CompilerParams(use_tc_tiling_on_sc=False),
    )
    def k(table_hbm, idx_hbm, out_hbm, idx_v, rows_v, sem):
        wid = lax.axis_index("s") * NC + lax.axis_index("c")
        base = wid * b_per_w
        for c in range(b_per_w // CH):
            off = base + c * CH
            pltpu.sync_copy(idx_hbm.at[pl.ds(off, CH)], idx_v)
            pltpu.async_copy(table_hbm.at[idx_v], rows_v, sem).wait()
            pltpu.sync_copy(rows_v, out_hbm.at[pl.ds(off, CH)])

    return k(table, idx)


GPB = 4


def _tc_body(x_ref, w_ref, bias_ref, wfc_ref, fcb_ref, o_ref, vals_ref):
    X = x_ref[...]
    zg = jnp.zeros((GPB, 1, 128), jnp.float32)
    X1 = jnp.concatenate([X[:, 1:], zg], axis=1)
    X2 = jnp.concatenate([X[:, 2:], zg, zg], axis=1)

    def mm(a, w):
        r = jnp.dot(a.reshape(GPB * L, 128), w,
                    preferred_element_type=jnp.float32)
        return r.reshape(GPB, L, 128)

    Y = mm(X, w_ref[0]) + mm(X1, w_ref[1]) + mm(X2, w_ref[2])
    Y = Y + bias_ref[0:1]
    s = lax.bitcast_convert_type(Y, jnp.int32)
    tio = lax.broadcasted_iota(jnp.int32, (GPB, L, 128), 1)
    low = jnp.where(s >= 0, jnp.int32(255) - tio, tio)
    keyf = lax.bitcast_convert_type((s & ~jnp.int32(255)) | low, jnp.float32)
    keyf = jnp.where(tio < LV, keyf, NEG)

    def step(k, keyf):
        m = jnp.max(keyf, axis=1, keepdims=True)
        vals_ref[:, pl.ds(k, 1)] = m
        return jnp.where(keyf == m, NEG, keyf)

    lax.fori_loop(0, K, step, keyf, unroll=True)
    b50 = lax.bitcast_convert_type(vals_ref[:, 0:K], jnp.int32)
    V50 = lax.bitcast_convert_type(
        (b50 & ~jnp.int32(255)) | jnp.int32(128), jnp.float32)
    out = jnp.dot(V50.reshape(GPB * K, 128), wfc_ref[...],
                  preferred_element_type=jnp.float32).reshape(GPB, K, 16)
    o_ref[...] = jnp.maximum(out + fcb_ref[0:1], 0.0)


def _tc_stage(xg, wbig, bias_t, wfc, fcb):
    return pl.pallas_call(
        _tc_body,
        grid=(G // GPB,),
        in_specs=[
            pl.BlockSpec((GPB, L, 128), lambda g: (g, 0, 0)),
            pl.BlockSpec((KW, 128, 128), lambda g: (0, 0, 0)),
            pl.BlockSpec((8, 128), lambda g: (0, 0)),
            pl.BlockSpec((128, 16), lambda g: (0, 0)),
            pl.BlockSpec((8, 16), lambda g: (0, 0)),
        ],
        out_specs=pl.BlockSpec((GPB, K, 16), lambda g: (g, 0, 0)),
        out_shape=jax.ShapeDtypeStruct((G, K, 16), jnp.float32),
        scratch_shapes=[pltpu.VMEM((GPB, K + 6, 128), jnp.float32)],
    )(xg, wbig, bias_t, wfc, fcb)


def kernel(inputs, emb, conv_w, conv_b, fc_w, fc_b):
    seqs = inputs.reshape(S, L).astype(jnp.int32)
    idx = seqs.reshape(G, GS, L).transpose(0, 2, 1).reshape(-1)
    table = jnp.concatenate(
        [emb, jnp.zeros((V, DP - D), jnp.float32)], axis=1)

    gathered = _sc_gather(table, idx)
    xg = gathered.reshape(G, L, GS * DP)

    eye8 = jnp.eye(GS, dtype=jnp.float32)
    wp = jnp.zeros((KW, DP, DP), jnp.float32).at[:, :D, :D].set(conv_w)
    wbig = jnp.stack([jnp.kron(eye8, wp[w]) for w in range(KW)])
    bias_t = jnp.tile(
        jnp.concatenate([conv_b, jnp.zeros((DP - F,), jnp.float32)]), (GS,))
    bias_t = jnp.broadcast_to(bias_t, (8, 128))
    wfc = jnp.kron(eye8, jnp.zeros((DP, 2), jnp.float32).at[:D].set(fc_w))
    fcb = jnp.broadcast_to(jnp.tile(fc_b, (GS,)), (8, 16))

    out = _tc_stage(xg, wbig, bias_t, wfc, fcb)

    out = out.reshape(G, K, GS, 2).transpose(0, 2, 1, 3)
    out = out.reshape(B, 2, K, 2).reshape(B, 2 * K, 2)
    return out


# speedup vs baseline: 19.9168x; 1.1569x over previous
import functools

import jax
import jax.numpy as jnp
from jax import lax
from jax.experimental import pallas as pl
from jax.experimental.pallas import tpu as pltpu

B, L, V, D, F, K, KW = 1024, 200, 100000, 10, 10, 50, 3
DP = 16
S = 2 * B
GS = 8
G = S // GS
N = S * L
LV = L - KW + 1
NEG = float(jnp.finfo(jnp.float32).min)


def _sc_gather(table, idx):
    from jax.experimental.pallas import tpu_sc as plsc

    info = plsc.get_sparse_core_info()
    NC, NS = info.num_cores, info.num_subcores
    NW = NC * NS
    b_per_w = N // NW
    CH = 3200
    mesh = plsc.VectorSubcoreMesh(core_axis_name="c", subcore_axis_name="s")

    @functools.partial(
        pl.kernel,
        mesh=mesh,
        out_type=jax.ShapeDtypeStruct((N, DP), jnp.float32),
        scratch_types=[
            pltpu.VMEM((CH,), jnp.int32),
            pltpu.VMEM((CH, DP), jnp.float32),
            pltpu.SemaphoreType.DMA,
        ],
        compiler_params=pltpu.CompilerParams(use_tc_tiling_on_sc=False),
    )
    def k(table_hbm, idx_hbm, out_hbm, idx_v, rows_v, sem):
        wid = lax.axis_index("s") * NC + lax.axis_index("c")
        base = wid * b_per_w
        for c in range(b_per_w // CH):
            off = base + c * CH
            pltpu.sync_copy(idx_hbm.at[pl.ds(off, CH)], idx_v)
            pltpu.async_copy(table_hbm.at[idx_v], rows_v, sem).wait()
            pltpu.sync_copy(rows_v, out_hbm.at[pl.ds(off, CH)])

    return k(table, idx)


GPB = 4


def _tc_body(x_ref, w_ref, bias_ref, wfc_ref, fcb_ref, o_ref, vals_ref):
    X = x_ref[...]
    zg = jnp.zeros((GPB, 1, 128), jnp.float32)
    X1 = jnp.concatenate([X[:, 1:], zg], axis=1)
    X2 = jnp.concatenate([X[:, 2:], zg, zg], axis=1)

    def mm(a, w):
        r = jnp.dot(a.reshape(GPB * L, 128), w,
                    preferred_element_type=jnp.float32)
        return r.reshape(GPB, L, 128)

    Y = mm(X, w_ref[0]) + mm(X1, w_ref[1]) + mm(X2, w_ref[2])
    Y = Y + bias_ref[0:1]
    s = lax.bitcast_convert_type(Y, jnp.int32)
    tio = lax.broadcasted_iota(jnp.int32, (GPB, L, 128), 1)
    low = jnp.where(s >= 0, jnp.int32(255) - tio, tio)
    keyf = lax.bitcast_convert_type((s & ~jnp.int32(255)) | low, jnp.float32)
    keyf = jnp.where(tio < LV, keyf, NEG)
    zpad = jnp.full((GPB, 8, 128), NEG, jnp.float32)
    keyp = jnp.concatenate([keyf, zpad], axis=1)
    A = keyp[:, :104]
    Bh = keyp[:, 104:]
    P = jnp.maximum(A, Bh)
    R = jnp.minimum(A, Bh)

    def step(k, carry):
        P, R = carry
        m = jnp.max(P, axis=1, keepdims=True)
        vals_ref[:, pl.ds(k, 1)] = m
        hit = P == m
        return jnp.where(hit, R, P), jnp.where(hit, NEG, R)

    lax.fori_loop(0, K, step, (P, R), unroll=True)
    b50 = lax.bitcast_convert_type(vals_ref[:, 0:K], jnp.int32)
    V50 = lax.bitcast_convert_type(
        (b50 & ~jnp.int32(255)) | jnp.int32(128), jnp.float32)
    out = jnp.dot(V50.reshape(GPB * K, 128), wfc_ref[...],
                  preferred_element_type=jnp.float32).reshape(GPB, K, 16)
    o_ref[...] = jnp.maximum(out + fcb_ref[0:1], 0.0)


def _tc_stage(xg, wbig, bias_t, wfc, fcb):
    return pl.pallas_call(
        _tc_body,
        grid=(G // GPB,),
        in_specs=[
            pl.BlockSpec((GPB, L, 128), lambda g: (g, 0, 0)),
            pl.BlockSpec((KW, 128, 128), lambda g: (0, 0, 0)),
            pl.BlockSpec((8, 128), lambda g: (0, 0)),
            pl.BlockSpec((128, 16), lambda g: (0, 0)),
            pl.BlockSpec((8, 16), lambda g: (0, 0)),
        ],
        out_specs=pl.BlockSpec((GPB, K, 16), lambda g: (g, 0, 0)),
        out_shape=jax.ShapeDtypeStruct((G, K, 16), jnp.float32),
        scratch_shapes=[pltpu.VMEM((GPB, K + 6, 128), jnp.float32)],
    )(xg, wbig, bias_t, wfc, fcb)


def kernel(inputs, emb, conv_w, conv_b, fc_w, fc_b):
    seqs = inputs.reshape(S, L).astype(jnp.int32)
    idx = seqs.reshape(G, GS, L).transpose(0, 2, 1).reshape(-1)
    table = jnp.concatenate(
        [emb, jnp.zeros((V, DP - D), jnp.float32)], axis=1)

    gathered = _sc_gather(table, idx)
    xg = gathered.reshape(G, L, GS * DP)

    eye8 = jnp.eye(GS, dtype=jnp.float32)
    wp = jnp.zeros((KW, DP, DP), jnp.float32).at[:, :D, :D].set(conv_w)
    wbig = jnp.stack([jnp.kron(eye8, wp[w]) for w in range(KW)])
    bias_t = jnp.tile(
        jnp.concatenate([conv_b, jnp.zeros((DP - F,), jnp.float32)]), (GS,))
    bias_t = jnp.broadcast_to(bias_t, (8, 128))
    wfc = jnp.kron(eye8, jnp.zeros((DP, 2), jnp.float32).at[:D].set(fc_w))
    fcb = jnp.broadcast_to(jnp.tile(fc_b, (GS,)), (8, 16))

    out = _tc_stage(xg, wbig, bias_t, wfc, fcb)

    out = out.reshape(G, K, GS, 2).transpose(0, 2, 1, 3)
    out = out.reshape(B, 2, K, 2).reshape(B, 2 * K, 2)
    return out


# speedup vs baseline: 21.9820x; 1.1037x over previous
import functools

import jax
import jax.numpy as jnp
from jax import lax
from jax.experimental import pallas as pl
from jax.experimental.pallas import tpu as pltpu

B, L, V, D, F, K, KW = 1024, 200, 100000, 10, 10, 50, 3
DP = 16
S = 2 * B
GS = 8
G = S // GS
N = S * L
LV = L - KW + 1
NEG = float(jnp.finfo(jnp.float32).min)


def _sc_gather(table, idx):
    from jax.experimental.pallas import tpu_sc as plsc

    info = plsc.get_sparse_core_info()
    NC, NS = info.num_cores, info.num_subcores
    NW = NC * NS
    b_per_w = N // NW
    CH = 3200
    mesh = plsc.VectorSubcoreMesh(core_axis_name="c", subcore_axis_name="s")

    @functools.partial(
        pl.kernel,
        mesh=mesh,
        out_type=jax.ShapeDtypeStruct((N, DP), jnp.float32),
        scratch_types=[
            pltpu.VMEM((CH,), jnp.int32),
            pltpu.VMEM((CH, DP), jnp.float32),
            pltpu.SemaphoreType.DMA,
        ],
        compiler_params=pltpu.CompilerParams(use_tc_tiling_on_sc=False),
    )
    def k(table_hbm, idx_hbm, out_hbm, idx_v, rows_v, sem):
        wid = lax.axis_index("s") * NC + lax.axis_index("c")
        base = wid * b_per_w
        for c in range(b_per_w // CH):
            off = base + c * CH
            pltpu.sync_copy(idx_hbm.at[pl.ds(off, CH)], idx_v)
            pltpu.async_copy(table_hbm.at[idx_v], rows_v, sem).wait()
            pltpu.sync_copy(rows_v, out_hbm.at[pl.ds(off, CH)])

    return k(table, idx)


GPB = 4


def _tc_body(x_ref, w_ref, bias_ref, wfc_ref, fcb_ref, o_ref, vals_ref):
    X = x_ref[...]
    zg = jnp.zeros((GPB, 1, 128), jnp.float32)
    X1 = jnp.concatenate([X[:, 1:], zg], axis=1)
    X2 = jnp.concatenate([X[:, 2:], zg, zg], axis=1)

    def mm(a, w):
        r = jnp.dot(a.reshape(GPB * L, 128), w,
                    preferred_element_type=jnp.float32)
        return r.reshape(GPB, L, 128)

    Y = mm(X, w_ref[0]) + mm(X1, w_ref[1]) + mm(X2, w_ref[2])
    Y = Y + bias_ref[0:1]
    s = lax.bitcast_convert_type(Y, jnp.int32)
    tio = lax.broadcasted_iota(jnp.int32, (GPB, L, 128), 1)
    low = jnp.where(s >= 0, jnp.int32(255) - tio, tio)
    keyf = lax.bitcast_convert_type((s & ~jnp.int32(255)) | low, jnp.float32)
    keyf = jnp.where(tio < LV, keyf, NEG)
    zpad = jnp.full((GPB, 24, 128), NEG, jnp.float32)
    keyp = jnp.concatenate([keyf, zpad], axis=1)
    a, b = keyp[:, :56], keyp[:, 56:112]
    c, d = keyp[:, 112:168], keyp[:, 168:]
    a, b = jnp.maximum(a, b), jnp.minimum(a, b)
    c, d = jnp.maximum(c, d), jnp.minimum(c, d)
    a, c = jnp.maximum(a, c), jnp.minimum(a, c)
    b, d = jnp.maximum(b, d), jnp.minimum(b, d)
    b, c = jnp.maximum(b, c), jnp.minimum(b, c)

    def step(k, carry):
        s0, s1, s2, s3 = carry
        m = jnp.max(s0, axis=1, keepdims=True)
        vals_ref[:, pl.ds(k, 1)] = m
        hit = s0 == m
        return (jnp.where(hit, s1, s0), jnp.where(hit, s2, s1),
                jnp.where(hit, s3, s2), jnp.where(hit, NEG, s3))

    lax.fori_loop(0, K, step, (a, b, c, d), unroll=True)
    b50 = lax.bitcast_convert_type(vals_ref[:, 0:K], jnp.int32)
    V50 = lax.bitcast_convert_type(
        (b50 & ~jnp.int32(255)) | jnp.int32(128), jnp.float32)
    out = jnp.dot(V50.reshape(GPB * K, 128), wfc_ref[...],
                  preferred_element_type=jnp.float32).reshape(GPB, K, 16)
    o_ref[...] = jnp.maximum(out + fcb_ref[0:1], 0.0)


def _tc_stage(xg, wbig, bias_t, wfc, fcb):
    return pl.pallas_call(
        _tc_body,
        grid=(G // GPB,),
        in_specs=[
            pl.BlockSpec((GPB, L, 128), lambda g: (g, 0, 0)),
            pl.BlockSpec((KW, 128, 128), lambda g: (0, 0, 0)),
            pl.BlockSpec((8, 128), lambda g: (0, 0)),
            pl.BlockSpec((128, 16), lambda g: (0, 0)),
            pl.BlockSpec((8, 16), lambda g: (0, 0)),
        ],
        out_specs=pl.BlockSpec((GPB, K, 16), lambda g: (g, 0, 0)),
        out_shape=jax.ShapeDtypeStruct((G, K, 16), jnp.float32),
        scratch_shapes=[pltpu.VMEM((GPB, K + 6, 128), jnp.float32)],
    )(xg, wbig, bias_t, wfc, fcb)


def kernel(inputs, emb, conv_w, conv_b, fc_w, fc_b):
    seqs = inputs.reshape(S, L).astype(jnp.int32)
    idx = seqs.reshape(G, GS, L).transpose(0, 2, 1).reshape(-1)
    table = jnp.concatenate(
        [emb, jnp.zeros((V, DP - D), jnp.float32)], axis=1)

    gathered = _sc_gather(table, idx)
    xg = gathered.reshape(G, L, GS * DP)

    eye8 = jnp.eye(GS, dtype=jnp.float32)
    wp = jnp.zeros((KW, DP, DP), jnp.float32).at[:, :D, :D].set(conv_w)
    wbig = jnp.stack([jnp.kron(eye8, wp[w]) for w in range(KW)])
    bias_t = jnp.tile(
        jnp.concatenate([conv_b, jnp.zeros((DP - F,), jnp.float32)]), (GS,))
    bias_t = jnp.broadcast_to(bias_t, (8, 128))
    wfc = jnp.kron(eye8, jnp.zeros((DP, 2), jnp.float32).at[:D].set(fc_w))
    fcb = jnp.broadcast_to(jnp.tile(fc_b, (GS,)), (8, 16))

    out = _tc_stage(xg, wbig, bias_t, wfc, fcb)

    out = out.reshape(G, K, GS, 2).transpose(0, 2, 1, 3)
    out = out.reshape(B, 2, K, 2).reshape(B, 2 * K, 2)
    return out
